# baseline (speedup 1.0000x reference)
import functools

import jax
import jax.numpy as jnp
from jax.experimental import pallas as pl
from jax.experimental.pallas import tpu as pltpu

_LANES = 128
_OUT_W = 8


def _tree_sum(parts):
    while len(parts) > 1:
        nxt = [parts[i] + parts[i + 1] for i in range(0, len(parts) - 1, 2)]
        if len(parts) % 2:
            nxt.append(parts[-1])
        parts = nxt
    return parts[0]


def _head_kernel(self_ref, nb_ref, inv_ref, w_ref, out_ref,
                 *, n_nb, hidden):
    bt = self_ref.shape[0]
    nb2 = nb_ref[...].reshape(bt * n_nb, hidden)
    acc_nb = jnp.dot(nb2, w_ref[hidden:2 * hidden, :],
                     preferred_element_type=jnp.float32)
    acc_nb = jnp.sum(acc_nb.reshape(bt, n_nb, _LANES), axis=1)

    acc = jnp.dot(self_ref[...], w_ref[0:hidden, :],
                  preferred_element_type=jnp.float32)
    acc += acc_nb * inv_ref[...]
    acc += w_ref[2 * hidden:2 * hidden + 1, :]
    out_ref[...] = acc[:, :_OUT_W]


def kernel(emb_self, emb_nb, weight, bias, nb_counts):
    B, H = emb_self.shape
    _, N, _ = emb_nb.shape
    half = N // 2

    wt = weight.astype(jnp.float32)
    w_pack = jnp.zeros((2 * H + 8, _LANES), jnp.float32)
    w_pack = w_pack.at[0:2 * H, :2].set(wt.T)
    w_pack = w_pack.at[2 * H, :2].set(bias.astype(jnp.float32))

    inv_cnt = (1.0 / jnp.maximum(nb_counts.astype(jnp.float32), 1.0)
               ).reshape(B, 1)

    bt = min(B, 128)
    grid = (pl.cdiv(B, bt),)

    out = pl.pallas_call(
        functools.partial(_head_kernel, n_nb=N, hidden=H),
        out_shape=jax.ShapeDtypeStruct((B, _OUT_W), jnp.float32),
        grid_spec=pl.GridSpec(
            grid=grid,
            in_specs=[
                pl.BlockSpec((bt, H), lambda i: (i, 0)),
                pl.BlockSpec((bt, N, H), lambda i: (i, 0, 0)),
                pl.BlockSpec((bt, 1), lambda i: (i, 0)),
                pl.BlockSpec((2 * H + 8, _LANES), lambda i: (0, 0)),
            ],
            out_specs=pl.BlockSpec((bt, _OUT_W), lambda i: (i, 0)),
        ),
        compiler_params=pltpu.CompilerParams(
            dimension_semantics=("parallel",),
            vmem_limit_bytes=64 << 20),
        cost_estimate=pl.CostEstimate(
            flops=2 * B * H * _LANES * 2 + B * (N - 1) * H,
            transcendentals=0,
            bytes_accessed=B * (N + 1) * H * 4 + B * 4 + B * _OUT_W * 4
                           + (2 * H + 8) * _LANES * 4),
    )(emb_self, emb_nb, inv_cnt, w_pack)

    return out[:, :2]


# speedup vs baseline: 1.2208x; 1.2208x over previous
import functools

import jax
import jax.numpy as jnp
from jax.experimental import pallas as pl
from jax.experimental.pallas import tpu as pltpu

_LANES = 128
_OUT_W = 8


def _tree_sum(parts):
    while len(parts) > 1:
        nxt = [parts[i] + parts[i + 1] for i in range(0, len(parts) - 1, 2)]
        if len(parts) % 2:
            nxt.append(parts[-1])
        parts = nxt
    return parts[0]


def _head_kernel(self_ref, nb_ref, inv_ref, w_ref, out_ref,
                 *, n_nb, hidden):
    bt = self_ref.shape[0]
    nb2 = nb_ref[...].reshape(bt * n_nb, hidden)
    acc_nb = jnp.dot(nb2, w_ref[hidden:2 * hidden, :],
                     preferred_element_type=jnp.float32)
    acc_nb = jnp.sum(acc_nb.reshape(bt, n_nb, _LANES), axis=1)

    acc = jnp.dot(self_ref[...], w_ref[0:hidden, :],
                  preferred_element_type=jnp.float32)
    acc += acc_nb * inv_ref[...]
    acc += w_ref[2 * hidden:2 * hidden + 1, :]
    out_ref[...] = acc[:, :_OUT_W]


def kernel(emb_self, emb_nb, weight, bias, nb_counts):
    B, H = emb_self.shape
    _, N, _ = emb_nb.shape
    half = N // 2

    wt = weight.astype(jnp.float32)
    w_pack = jnp.zeros((2 * H + 8, _LANES), jnp.float32)
    w_pack = w_pack.at[0:2 * H, :2].set(wt.T)
    w_pack = w_pack.at[2 * H, :2].set(bias.astype(jnp.float32))

    inv_cnt = (1.0 / jnp.maximum(nb_counts.astype(jnp.float32), 1.0)
               ).reshape(B, 1)

    bt = min(B, 384)
    grid = (pl.cdiv(B, bt),)

    out = pl.pallas_call(
        functools.partial(_head_kernel, n_nb=N, hidden=H),
        out_shape=jax.ShapeDtypeStruct((B, _OUT_W), jnp.float32),
        grid_spec=pl.GridSpec(
            grid=grid,
            in_specs=[
                pl.BlockSpec((bt, H), lambda i: (i, 0)),
                pl.BlockSpec((bt, N, H), lambda i: (i, 0, 0)),
                pl.BlockSpec((bt, 1), lambda i: (i, 0)),
                pl.BlockSpec((2 * H + 8, _LANES), lambda i: (0, 0)),
            ],
            out_specs=pl.BlockSpec((bt, _OUT_W), lambda i: (i, 0)),
        ),
        compiler_params=pltpu.CompilerParams(
            dimension_semantics=("parallel",),
            vmem_limit_bytes=64 << 20),
        cost_estimate=pl.CostEstimate(
            flops=2 * B * H * _LANES * 2 + B * (N - 1) * H,
            transcendentals=0,
            bytes_accessed=B * (N + 1) * H * 4 + B * 4 + B * _OUT_W * 4
                           + (2 * H + 8) * _LANES * 4),
    )(emb_self, emb_nb, inv_cnt, w_pack)

    return out[:, :2]


# speedup vs baseline: 1.2250x; 1.0034x over previous
import functools

import jax
import jax.numpy as jnp
from jax.experimental import pallas as pl
from jax.experimental.pallas import tpu as pltpu

_LANES = 128
_OUT_W = 8


def _tree_sum(parts):
    while len(parts) > 1:
        nxt = [parts[i] + parts[i + 1] for i in range(0, len(parts) - 1, 2)]
        if len(parts) % 2:
            nxt.append(parts[-1])
        parts = nxt
    return parts[0]


def _head_kernel(self_ref, nb_a_ref, nb_b_ref, inv_ref, w_ref, out_ref,
                 *, n_nb, hidden):
    hb = nb_a_ref.shape[0]
    wn = w_ref[hidden:2 * hidden, :]
    acc_a = jnp.dot(nb_a_ref[...].reshape(hb * n_nb, hidden), wn,
                    preferred_element_type=jnp.float32)
    acc_b = jnp.dot(nb_b_ref[...].reshape(hb * n_nb, hidden), wn,
                    preferred_element_type=jnp.float32)
    acc_nb = jnp.sum(
        jnp.concatenate([acc_a, acc_b], axis=0).reshape(-1, n_nb, _LANES),
        axis=1)

    acc = jnp.dot(self_ref[...], w_ref[0:hidden, :],
                  preferred_element_type=jnp.float32)
    acc += acc_nb * inv_ref[...]
    acc += w_ref[2 * hidden:2 * hidden + 1, :]
    out_ref[...] = acc[:, :_OUT_W]


def kernel(emb_self, emb_nb, weight, bias, nb_counts):
    B, H = emb_self.shape
    _, N, _ = emb_nb.shape
    half = N // 2

    wt = weight.astype(jnp.float32)
    w_pack = jnp.zeros((2 * H + 8, _LANES), jnp.float32)
    w_pack = w_pack.at[0:2 * H, :2].set(wt.T)
    w_pack = w_pack.at[2 * H, :2].set(bias.astype(jnp.float32))

    inv_cnt = (1.0 / jnp.maximum(nb_counts.astype(jnp.float32), 1.0)
               ).reshape(B, 1)

    bt = min(B, 352)
    grid = (pl.cdiv(B, bt),)

    out = pl.pallas_call(
        functools.partial(_head_kernel, n_nb=N, hidden=H),
        out_shape=jax.ShapeDtypeStruct((B, _OUT_W), jnp.float32),
        grid_spec=pl.GridSpec(
            grid=grid,
            in_specs=[
                pl.BlockSpec((bt, H), lambda i: (i, 0)),
                pl.BlockSpec((bt // 2, N, H), lambda i: (2 * i, 0, 0)),
                pl.BlockSpec((bt // 2, N, H), lambda i: (2 * i + 1, 0, 0)),
                pl.BlockSpec((bt, 1), lambda i: (i, 0)),
                pl.BlockSpec((2 * H + 8, _LANES), lambda i: (0, 0)),
            ],
            out_specs=pl.BlockSpec((bt, _OUT_W), lambda i: (i, 0)),
        ),
        compiler_params=pltpu.CompilerParams(
            dimension_semantics=("parallel",),
            vmem_limit_bytes=64 << 20),
        cost_estimate=pl.CostEstimate(
            flops=2 * B * H * _LANES * 2 + B * (N - 1) * H,
            transcendentals=0,
            bytes_accessed=B * (N + 1) * H * 4 + B * 4 + B * _OUT_W * 4
                           + (2 * H + 8) * _LANES * 4),
    )(emb_self, emb_nb, emb_nb, inv_cnt, w_pack)

    return out[:, :2]


# speedup vs baseline: 1.2338x; 1.0072x over previous
import functools

import jax
import jax.numpy as jnp
from jax.experimental import pallas as pl
from jax.experimental.pallas import tpu as pltpu

_LANES = 128
_OUT_W = 8


def _tree_sum(parts):
    while len(parts) > 1:
        nxt = [parts[i] + parts[i + 1] for i in range(0, len(parts) - 1, 2)]
        if len(parts) % 2:
            nxt.append(parts[-1])
        parts = nxt
    return parts[0]


def _head_kernel(self_ref, nb_ref, inv_ref, w_ref, out_ref,
                 *, n_nb, hidden):
    bt = self_ref.shape[0]
    nb2 = nb_ref[...].reshape(bt * n_nb, hidden)
    acc_nb = jnp.dot(nb2, w_ref[hidden:2 * hidden, :],
                     preferred_element_type=jnp.float32)
    acc_nb = jnp.sum(acc_nb.reshape(bt, n_nb, _LANES), axis=1)

    acc = jnp.dot(self_ref[...], w_ref[0:hidden, :],
                  preferred_element_type=jnp.float32)
    acc += acc_nb * inv_ref[...]
    acc += w_ref[2 * hidden:2 * hidden + 1, :]
    out_ref[...] = acc[:, :_OUT_W]


def kernel(emb_self, emb_nb, weight, bias, nb_counts):
    B, H = emb_self.shape
    _, N, _ = emb_nb.shape
    half = N // 2

    wt = weight.astype(jnp.float32)
    w_pack = jnp.zeros((2 * H + 8, _LANES), jnp.float32)
    w_pack = w_pack.at[0:2 * H, :2].set(wt.T)
    w_pack = w_pack.at[2 * H, :2].set(bias.astype(jnp.float32))

    inv_cnt = (1.0 / jnp.maximum(nb_counts.astype(jnp.float32), 1.0)
               ).reshape(B, 1)

    bt = min(B, 320)
    grid = (pl.cdiv(B, bt),)

    out = pl.pallas_call(
        functools.partial(_head_kernel, n_nb=N, hidden=H),
        out_shape=jax.ShapeDtypeStruct((B, _OUT_W), jnp.float32),
        grid_spec=pl.GridSpec(
            grid=grid,
            in_specs=[
                pl.BlockSpec((bt, H), lambda i: (i, 0)),
                pl.BlockSpec((bt, N, H), lambda i: (i, 0, 0)),
                pl.BlockSpec((bt, 1), lambda i: (i, 0)),
                pl.BlockSpec((2 * H + 8, _LANES), lambda i: (0, 0)),
            ],
            out_specs=pl.BlockSpec((bt, _OUT_W), lambda i: (i, 0)),
        ),
        compiler_params=pltpu.CompilerParams(
            dimension_semantics=("parallel",),
            vmem_limit_bytes=64 << 20),
        cost_estimate=pl.CostEstimate(
            flops=2 * B * H * _LANES * 2 + B * (N - 1) * H,
            transcendentals=0,
            bytes_accessed=B * (N + 1) * H * 4 + B * 4 + B * _OUT_W * 4
                           + (2 * H + 8) * _LANES * 4),
    )(emb_self, emb_nb, inv_cnt, w_pack)

    return out[:, :2]
